# baseline (speedup 1.0000x reference)
import functools

import jax
import jax.numpy as jnp
import numpy as np
from jax import lax
from jax.experimental import pallas as pl
from jax.experimental.pallas import tpu as pltpu
from jax.experimental.pallas import tpu_sc as plsc

N = 10000
D = 128
K = 8
BLK = 128
NP = 10240
NBLK = NP // BLK
WALK_STEPS = 3

CH = 2000
NCH = N // CH
CW = K * CH
M16 = CH // 16


def _knn_coin_body(en_ref, enT_ref, emb_ref, W1a_ref, b1e_ref, W2p_ref,
                   b2p_ref, nbr_ref, g_ref):
    i = pl.program_id(0)
    en_blk = en_ref[...]
    enT = enT_ref[...]
    sim = jax.lax.dot_general(
        en_blk, enT, (((1,), (0,)), ((), ())),
        precision=jax.lax.Precision.DEFAULT,
        preferred_element_type=jnp.float32)
    colf = jax.lax.broadcasted_iota(jnp.int32, (BLK, NP), 1).astype(
        jnp.float32)
    rowf = jnp.float32(i * BLK) + jax.lax.broadcasted_iota(
        jnp.int32, (BLK, NP), 0).astype(jnp.float32)
    neg = jnp.float32(-jnp.inf)
    sim = jnp.where((colf == rowf) | (colf >= N), neg, sim)
    idxs = []
    big = jnp.float32(NP + 1)
    colw = jnp.concatenate(
        [jax.lax.broadcasted_iota(jnp.int32, (NP, 1), 0).astype(jnp.float32),
         jnp.ones((NP, 1), jnp.float32)], axis=1)
    for _ in range(K):
        m = jnp.max(sim, axis=1, keepdims=True)
        eqf = (sim == m).astype(jnp.float32)
        cs = jax.lax.dot_general(eqf, colw, (((1,), (0,)), ((), ())),
                                 precision=jax.lax.Precision.DEFAULT,
                                 preferred_element_type=jnp.float32)
        csum, cnt = cs[:, :1], cs[:, 1:2]
        am = jax.lax.cond(
            jnp.max(cnt) > 1.5,
            lambda: jnp.min(jnp.where(sim == m, colf, big), axis=1,
                            keepdims=True),
            lambda: csum)
        idxs.append(am)
        sim = jnp.where(colf == am, neg, sim)
    nbr_ref[...] = jnp.concatenate(idxs, axis=1).astype(jnp.int32)

    emb_blk = emb_ref[...]
    h = jnp.maximum(
        jax.lax.dot_general(emb_blk, W1a_ref[...], (((1,), (0,)), ((), ())),
                            precision=jax.lax.Precision.DEFAULT,
                            preferred_element_type=jnp.float32)
        + b1e_ref[...], 0.0)
    a = jax.lax.dot_general(h, W2p_ref[...], (((1,), (0,)), ((), ())),
                            precision=jax.lax.Precision.DEFAULT,
                            preferred_element_type=jnp.float32) + b2p_ref[...]
    a8 = a[:, :K]
    n2 = jnp.sum(a8 * a8, axis=1, keepdims=True)
    safe = jnp.where(n2 > 0, n2, 1.0)
    g_ref[...] = jnp.where(n2 > 0, a8 * jax.lax.rsqrt(safe),
                           jnp.float32(1.0 / np.sqrt(K)))


@jax.jit
def _knn_coin(en_pad, emb_pad, W1a, b1e, W2p, b2p):
    enT = en_pad.T
    grid = (NBLK,)
    nbrs, g = pl.pallas_call(
        _knn_coin_body,
        grid=grid,
        in_specs=[
            pl.BlockSpec((BLK, D), lambda i: (i, 0)),
            pl.BlockSpec((D, NP), lambda i: (0, 0)),
            pl.BlockSpec((BLK, D), lambda i: (i, 0)),
            pl.BlockSpec((D, D), lambda i: (0, 0)),
            pl.BlockSpec((1, D), lambda i: (0, 0)),
            pl.BlockSpec((D, D), lambda i: (0, 0)),
            pl.BlockSpec((1, D), lambda i: (0, 0)),
        ],
        out_specs=[
            pl.BlockSpec((BLK, K), lambda i: (i, 0)),
            pl.BlockSpec((BLK, K), lambda i: (i, 0)),
        ],
        out_shape=[
            jax.ShapeDtypeStruct((NP, K), jnp.int32),
            jax.ShapeDtypeStruct((NP, K), jnp.float32),
        ],
    )(en_pad, enT, emb_pad, W1a, b1e, W2p, b2p)
    return nbrs, g


_SC_MESH = plsc.VectorSubcoreMesh(core_axis_name="c", subcore_axis_name="s")


@functools.partial(
    pl.kernel, mesh=_SC_MESH,
    compiler_params=pltpu.CompilerParams(needs_layout_passes=False),
    out_type=jax.ShapeDtypeStruct((N,), jnp.float32),
    scratch_types=[
        pltpu.VMEM((N * K,), jnp.float32),
        pltpu.VMEM((N,), jnp.float32),
        pltpu.VMEM((CW,), jnp.float32),
        pltpu.VMEM((CW,), jnp.int32),
    ],
)
def _sc_walk(g_hbm, idx_hbm, out_hbm, stv, d_buf, g_buf, idx_buf):
    out_buf = d_buf
    cid = lax.axis_index("c")
    sid = lax.axis_index("s")
    zeros16 = jnp.zeros((16,), jnp.float32)

    @pl.when((cid == 0) & (sid == 0))
    def _():
        c0 = jnp.full((16,), 1.0 / np.sqrt(N * K), jnp.float32)

        def fill(i, _):
            stv[pl.ds(i * 16, 16)] = c0
            return 0
        lax.fori_loop(0, N * K // 16, fill, 0)

        for step in range(WALK_STEPS):
            def dot_chunk(c, _):
                pltpu.sync_copy(g_hbm.at[c], g_buf)

                def m_body(m, _):
                    d = zeros16
                    for k in range(K):
                        d += (g_buf[pl.ds(k * CH + m * 16, 16)]
                              * stv[pl.ds(c * CW + k * CH + m * 16, 16)])
                    d_buf[pl.ds(c * CH + m * 16, 16)] = d
                    return 0
                lax.fori_loop(0, M16, m_body, 0)
                return 0
            lax.fori_loop(0, NCH, dot_chunk, 0)

            def zero_all(i, _):
                stv[pl.ds(i * 16, 16)] = zeros16
                return 0
            lax.fori_loop(0, N * K // 16, zero_all, 0)

            def scat_chunk(c, _):
                pltpu.sync_copy(g_hbm.at[c], g_buf)
                pltpu.sync_copy(idx_hbm.at[c], idx_buf)

                def m_body(m, _):
                    d = d_buf[pl.ds(c * CH + m * 16, 16)]
                    for k in range(K):
                        plsc.addupdate_scatter(
                            stv, [idx_buf[pl.ds(k * CH + m * 16, 16)]],
                            g_buf[pl.ds(k * CH + m * 16, 16)] * d)
                    return 0
                lax.fori_loop(0, M16, m_body, 0)
                return 0
            lax.fori_loop(0, NCH, scat_chunk, 0)

        def fin_c(c, acc):
            def fin_m(m, acc):
                av = zeros16
                for k in range(K):
                    v = stv[pl.ds(c * CW + k * CH + m * 16, 16)]
                    av += jnp.abs(v)
                    acc = acc + v * v
                out_buf[pl.ds(c * CH + m * 16, 16)] = av
                return acc
            return lax.fori_loop(0, M16, fin_m, acc)
        acc = lax.fori_loop(0, NCH, fin_c, zeros16)
        s_tot = jnp.sum(acc)
        sv = jnp.full((16,), s_tot, jnp.float32)
        seed = lax.bitcast_convert_type(
            jnp.full((16,), 0x5F3759DF, jnp.int32)
            - lax.shift_right_logical(
                lax.bitcast_convert_type(sv, jnp.int32), 1),
            jnp.float32)
        y = seed
        for _ in range(4):
            y = y * (1.5 - 0.5 * sv * y * y)
        uni = jnp.full((16,), np.sqrt(K / N), jnp.float32)
        pos = sv > 0

        def scale_m(m, _):
            o = out_buf[pl.ds(m * 16, 16)]
            out_buf[pl.ds(m * 16, 16)] = jnp.where(pos, o * y, uni)
            return 0
        lax.fori_loop(0, N // 16, scale_m, 0)
        pltpu.sync_copy(out_buf, out_hbm)


def _to_blocked(x):
    return x.T.reshape(K, NCH, CH).transpose(1, 0, 2).reshape(NCH, CW)


def kernel(emb, qv, W1, b1, W2, b2):
    en = emb / (jnp.linalg.norm(emb, axis=1, keepdims=True) + 1e-12)
    en_pad = jnp.pad(en, ((0, NP - N), (0, 0)))
    emb_pad = jnp.pad(emb, ((0, NP - N), (0, 0)))
    b1e = (b1 + qv @ W1[D:]).reshape(1, D)
    W1a = W1[:D]
    W2p = jnp.pad(W2, ((0, 0), (0, D - K)))
    b2p = jnp.pad(b2, (0, D - K)).reshape(1, D)
    nbrs_p, g_p = _knn_coin(en_pad, emb_pad, W1a, b1e, W2p, b2p)
    nbrs = nbrs_p[:N]
    g = g_p[:N]

    kcol = jnp.arange(K, dtype=jnp.int32)[None, :]
    dest = (nbrs // CH) * CW + kcol * CH + nbrs % CH
    g_b = _to_blocked(g)
    idx_b = _to_blocked(dest)
    return _sc_walk(g_b, idx_b)


# speedup vs baseline: 1.3777x; 1.3777x over previous
import functools

import jax
import jax.numpy as jnp
import numpy as np
from jax import lax
from jax.experimental import pallas as pl
from jax.experimental.pallas import tpu as pltpu
from jax.experimental.pallas import tpu_sc as plsc

N = 10000
D = 128
K = 8
BLK = 128
NP = 10240
NBLK = NP // BLK
WALK_STEPS = 3

CH = 2000
NCH = N // CH
CW = K * CH
M16 = CH // 16


def _knn_coin_body(en_ref, enT_ref, emb_ref, W1a_ref, b1e_ref, W2p_ref,
                   b2p_ref, nbr_ref, g_ref):
    i = pl.program_id(0)
    en_blk = en_ref[...]
    enT = enT_ref[...]
    sim = jax.lax.dot_general(
        en_blk, enT, (((1,), (0,)), ((), ())),
        precision=jax.lax.Precision.DEFAULT,
        preferred_element_type=jnp.float32)
    colf = jax.lax.broadcasted_iota(jnp.int32, (BLK, NP), 1).astype(
        jnp.float32)
    rowf = jnp.float32(i * BLK) + jax.lax.broadcasted_iota(
        jnp.int32, (BLK, NP), 0).astype(jnp.float32)
    neg = jnp.float32(-jnp.inf)
    sim = jnp.where((colf == rowf) | (colf >= N), neg, sim)
    idxs = []
    big = jnp.float32(NP + 1)
    for _ in range(K):
        m = jnp.max(sim, axis=1, keepdims=True)
        am = jnp.min(jnp.where(sim == m, colf, big), axis=1,
                     keepdims=True)
        idxs.append(am)
        sim = jnp.where(colf == am, neg, sim)
    nbr_ref[...] = jnp.concatenate(idxs, axis=1).astype(jnp.int32)

    emb_blk = emb_ref[...]
    h = jnp.maximum(
        jax.lax.dot_general(emb_blk, W1a_ref[...], (((1,), (0,)), ((), ())),
                            precision=jax.lax.Precision.DEFAULT,
                            preferred_element_type=jnp.float32)
        + b1e_ref[...], 0.0)
    a = jax.lax.dot_general(h, W2p_ref[...], (((1,), (0,)), ((), ())),
                            precision=jax.lax.Precision.DEFAULT,
                            preferred_element_type=jnp.float32) + b2p_ref[...]
    a8 = a[:, :K]
    n2 = jnp.sum(a8 * a8, axis=1, keepdims=True)
    safe = jnp.where(n2 > 0, n2, 1.0)
    g_ref[...] = jnp.where(n2 > 0, a8 * jax.lax.rsqrt(safe),
                           jnp.float32(1.0 / np.sqrt(K)))


@jax.jit
def _knn_coin(en_pad, emb_pad, W1a, b1e, W2p, b2p):
    enT = en_pad.T
    grid = (NBLK,)
    nbrs, g = pl.pallas_call(
        _knn_coin_body,
        grid=grid,
        in_specs=[
            pl.BlockSpec((BLK, D), lambda i: (i, 0)),
            pl.BlockSpec((D, NP), lambda i: (0, 0)),
            pl.BlockSpec((BLK, D), lambda i: (i, 0)),
            pl.BlockSpec((D, D), lambda i: (0, 0)),
            pl.BlockSpec((1, D), lambda i: (0, 0)),
            pl.BlockSpec((D, D), lambda i: (0, 0)),
            pl.BlockSpec((1, D), lambda i: (0, 0)),
        ],
        out_specs=[
            pl.BlockSpec((BLK, K), lambda i: (i, 0)),
            pl.BlockSpec((BLK, K), lambda i: (i, 0)),
        ],
        out_shape=[
            jax.ShapeDtypeStruct((NP, K), jnp.int32),
            jax.ShapeDtypeStruct((NP, K), jnp.float32),
        ],
    )(en_pad, enT, emb_pad, W1a, b1e, W2p, b2p)
    return nbrs, g


_SC_MESH = plsc.VectorSubcoreMesh(core_axis_name="c", subcore_axis_name="s")


@functools.partial(
    pl.kernel, mesh=_SC_MESH,
    compiler_params=pltpu.CompilerParams(needs_layout_passes=False),
    out_type=jax.ShapeDtypeStruct((N,), jnp.float32),
    scratch_types=[
        pltpu.VMEM((N * K,), jnp.float32),
        pltpu.VMEM((N,), jnp.float32),
        pltpu.VMEM((CW,), jnp.float32),
        pltpu.VMEM((CW,), jnp.int32),
    ],
)
def _sc_walk(g_hbm, idx_hbm, out_hbm, stv, d_buf, g_buf, idx_buf):
    out_buf = d_buf
    cid = lax.axis_index("c")
    sid = lax.axis_index("s")
    zeros16 = jnp.zeros((16,), jnp.float32)

    @pl.when((cid == 0) & (sid == 0))
    def _():
        c0 = jnp.full((16,), 1.0 / np.sqrt(N * K), jnp.float32)

        def fill(i, _):
            stv[pl.ds(i * 16, 16)] = c0
            return 0
        lax.fori_loop(0, N * K // 16, fill, 0)

        for step in range(WALK_STEPS):
            def dot_chunk(c, _):
                pltpu.sync_copy(g_hbm.at[c], g_buf)

                def m_body(m, _):
                    d = zeros16
                    for k in range(K):
                        d += (g_buf[pl.ds(k * CH + m * 16, 16)]
                              * stv[pl.ds(c * CW + k * CH + m * 16, 16)])
                    d_buf[pl.ds(c * CH + m * 16, 16)] = d
                    return 0
                lax.fori_loop(0, M16, m_body, 0)
                return 0
            lax.fori_loop(0, NCH, dot_chunk, 0)

            def zero_all(i, _):
                stv[pl.ds(i * 16, 16)] = zeros16
                return 0
            lax.fori_loop(0, N * K // 16, zero_all, 0)

            def scat_chunk(c, _):
                pltpu.sync_copy(g_hbm.at[c], g_buf)
                pltpu.sync_copy(idx_hbm.at[c], idx_buf)

                def m_body(m, _):
                    d = d_buf[pl.ds(c * CH + m * 16, 16)]
                    for k in range(K):
                        plsc.addupdate_scatter(
                            stv, [idx_buf[pl.ds(k * CH + m * 16, 16)]],
                            g_buf[pl.ds(k * CH + m * 16, 16)] * d)
                    return 0
                lax.fori_loop(0, M16, m_body, 0)
                return 0
            lax.fori_loop(0, NCH, scat_chunk, 0)

        def fin_c(c, acc):
            def fin_m(m, acc):
                av = zeros16
                for k in range(K):
                    v = stv[pl.ds(c * CW + k * CH + m * 16, 16)]
                    av += jnp.abs(v)
                    acc = acc + v * v
                out_buf[pl.ds(c * CH + m * 16, 16)] = av
                return acc
            return lax.fori_loop(0, M16, fin_m, acc)
        acc = lax.fori_loop(0, NCH, fin_c, zeros16)
        s_tot = jnp.sum(acc)
        sv = jnp.full((16,), s_tot, jnp.float32)
        seed = lax.bitcast_convert_type(
            jnp.full((16,), 0x5F3759DF, jnp.int32)
            - lax.shift_right_logical(
                lax.bitcast_convert_type(sv, jnp.int32), 1),
            jnp.float32)
        y = seed
        for _ in range(4):
            y = y * (1.5 - 0.5 * sv * y * y)
        uni = jnp.full((16,), np.sqrt(K / N), jnp.float32)
        pos = sv > 0

        def scale_m(m, _):
            o = out_buf[pl.ds(m * 16, 16)]
            out_buf[pl.ds(m * 16, 16)] = jnp.where(pos, o * y, uni)
            return 0
        lax.fori_loop(0, N // 16, scale_m, 0)
        pltpu.sync_copy(out_buf, out_hbm)


def _to_blocked(x):
    return x.T.reshape(K, NCH, CH).transpose(1, 0, 2).reshape(NCH, CW)


def kernel(emb, qv, W1, b1, W2, b2):
    en = emb / (jnp.linalg.norm(emb, axis=1, keepdims=True) + 1e-12)
    en_pad = jnp.pad(en, ((0, NP - N), (0, 0)))
    emb_pad = jnp.pad(emb, ((0, NP - N), (0, 0)))
    b1e = (b1 + qv @ W1[D:]).reshape(1, D)
    W1a = W1[:D]
    W2p = jnp.pad(W2, ((0, 0), (0, D - K)))
    b2p = jnp.pad(b2, (0, D - K)).reshape(1, D)
    nbrs_p, g_p = _knn_coin(en_pad, emb_pad, W1a, b1e, W2p, b2p)
    nbrs = nbrs_p[:N]
    g = g_p[:N]

    kcol = jnp.arange(K, dtype=jnp.int32)[None, :]
    dest = (nbrs // CH) * CW + kcol * CH + nbrs % CH
    g_b = _to_blocked(g)
    idx_b = _to_blocked(dest)
    return _sc_walk(g_b, idx_b)


# speedup vs baseline: 1.3788x; 1.0008x over previous
import functools

import jax
import jax.numpy as jnp
import numpy as np
from jax import lax
from jax.experimental import pallas as pl
from jax.experimental.pallas import tpu as pltpu
from jax.experimental.pallas import tpu_sc as plsc

N = 10000
D = 128
K = 8
BLK = 128
NP = 10240
NBLK = NP // BLK
WALK_STEPS = 3

CH = 2000
NCH = N // CH
CW = K * CH
M16 = CH // 16


def _knn_coin_body(en_ref, enT_ref, emb_ref, W1a_ref, b1e_ref, W2p_ref,
                   b2p_ref, nbr_ref, g_ref):
    i = pl.program_id(0)
    en_blk = en_ref[...]
    enT = enT_ref[...]
    sim = jax.lax.dot_general(
        en_blk, enT, (((1,), (0,)), ((), ())),
        precision=jax.lax.Precision.DEFAULT,
        preferred_element_type=jnp.float32)
    colf = jax.lax.broadcasted_iota(jnp.int32, (BLK, NP), 1).astype(
        jnp.float32)
    rowf = jnp.float32(i * BLK) + jax.lax.broadcasted_iota(
        jnp.int32, (BLK, NP), 0).astype(jnp.float32)
    neg = jnp.float32(-jnp.inf)
    sim = jnp.where((colf == rowf) | (colf >= N), neg, sim)
    idxs = []
    big = jnp.float32(NP + 1)
    for _ in range(K):
        m = jnp.max(sim, axis=1, keepdims=True)
        am = jnp.min(jnp.where(sim == m, colf, big), axis=1,
                     keepdims=True)
        idxs.append(am)
        sim = jnp.where(colf == am, neg, sim)
    nbr_ref[...] = jnp.concatenate(idxs, axis=1).astype(jnp.int32)

    emb_blk = emb_ref[...]
    h = jnp.maximum(
        jax.lax.dot_general(emb_blk, W1a_ref[...], (((1,), (0,)), ((), ())),
                            precision=jax.lax.Precision.DEFAULT,
                            preferred_element_type=jnp.float32)
        + b1e_ref[...], 0.0)
    a = jax.lax.dot_general(h, W2p_ref[...], (((1,), (0,)), ((), ())),
                            precision=jax.lax.Precision.DEFAULT,
                            preferred_element_type=jnp.float32) + b2p_ref[...]
    a8 = a[:, :K]
    n2 = jnp.sum(a8 * a8, axis=1, keepdims=True)
    safe = jnp.where(n2 > 0, n2, 1.0)
    g_ref[...] = jnp.where(n2 > 0, a8 * jax.lax.rsqrt(safe),
                           jnp.float32(1.0 / np.sqrt(K)))


@jax.jit
def _knn_coin(en_pad, emb_pad, W1a, b1e, W2p, b2p):
    enT = en_pad.T
    grid = (NBLK,)
    nbrs, g = pl.pallas_call(
        _knn_coin_body,
        grid=grid,
        compiler_params=pltpu.CompilerParams(
            dimension_semantics=("parallel",)),
        in_specs=[
            pl.BlockSpec((BLK, D), lambda i: (i, 0)),
            pl.BlockSpec((D, NP), lambda i: (0, 0)),
            pl.BlockSpec((BLK, D), lambda i: (i, 0)),
            pl.BlockSpec((D, D), lambda i: (0, 0)),
            pl.BlockSpec((1, D), lambda i: (0, 0)),
            pl.BlockSpec((D, D), lambda i: (0, 0)),
            pl.BlockSpec((1, D), lambda i: (0, 0)),
        ],
        out_specs=[
            pl.BlockSpec((BLK, K), lambda i: (i, 0)),
            pl.BlockSpec((BLK, K), lambda i: (i, 0)),
        ],
        out_shape=[
            jax.ShapeDtypeStruct((NP, K), jnp.int32),
            jax.ShapeDtypeStruct((NP, K), jnp.float32),
        ],
    )(en_pad, enT, emb_pad, W1a, b1e, W2p, b2p)
    return nbrs, g


_SC_MESH = plsc.VectorSubcoreMesh(core_axis_name="c", subcore_axis_name="s")


@functools.partial(
    pl.kernel, mesh=_SC_MESH,
    compiler_params=pltpu.CompilerParams(needs_layout_passes=False),
    out_type=jax.ShapeDtypeStruct((N,), jnp.float32),
    scratch_types=[
        pltpu.VMEM((N * K,), jnp.float32),
        pltpu.VMEM((N,), jnp.float32),
        pltpu.VMEM((CW,), jnp.float32),
        pltpu.VMEM((CW,), jnp.int32),
    ],
)
def _sc_walk(g_hbm, idx_hbm, out_hbm, stv, d_buf, g_buf, idx_buf):
    out_buf = d_buf
    cid = lax.axis_index("c")
    sid = lax.axis_index("s")
    zeros16 = jnp.zeros((16,), jnp.float32)

    @pl.when((cid == 0) & (sid == 0))
    def _():
        c0 = jnp.full((16,), 1.0 / np.sqrt(N * K), jnp.float32)

        def fill(i, _):
            stv[pl.ds(i * 16, 16)] = c0
            return 0
        lax.fori_loop(0, N * K // 16, fill, 0)

        for step in range(WALK_STEPS):
            def dot_chunk(c, _):
                pltpu.sync_copy(g_hbm.at[c], g_buf)

                def m_body(m, _):
                    d = zeros16
                    for k in range(K):
                        d += (g_buf[pl.ds(k * CH + m * 16, 16)]
                              * stv[pl.ds(c * CW + k * CH + m * 16, 16)])
                    d_buf[pl.ds(c * CH + m * 16, 16)] = d
                    return 0
                lax.fori_loop(0, M16, m_body, 0)
                return 0
            lax.fori_loop(0, NCH, dot_chunk, 0)

            def zero_all(i, _):
                stv[pl.ds(i * 16, 16)] = zeros16
                return 0
            lax.fori_loop(0, N * K // 16, zero_all, 0)

            def scat_chunk(c, _):
                pltpu.sync_copy(g_hbm.at[c], g_buf)
                pltpu.sync_copy(idx_hbm.at[c], idx_buf)

                def m_body(m, _):
                    d = d_buf[pl.ds(c * CH + m * 16, 16)]
                    for k in range(K):
                        plsc.addupdate_scatter(
                            stv, [idx_buf[pl.ds(k * CH + m * 16, 16)]],
                            g_buf[pl.ds(k * CH + m * 16, 16)] * d)
                    return 0
                lax.fori_loop(0, M16, m_body, 0)
                return 0
            lax.fori_loop(0, NCH, scat_chunk, 0)

        def fin_c(c, acc):
            def fin_m(m, acc):
                av = zeros16
                for k in range(K):
                    v = stv[pl.ds(c * CW + k * CH + m * 16, 16)]
                    av += jnp.abs(v)
                    acc = acc + v * v
                out_buf[pl.ds(c * CH + m * 16, 16)] = av
                return acc
            return lax.fori_loop(0, M16, fin_m, acc)
        acc = lax.fori_loop(0, NCH, fin_c, zeros16)
        s_tot = jnp.sum(acc)
        sv = jnp.full((16,), s_tot, jnp.float32)
        seed = lax.bitcast_convert_type(
            jnp.full((16,), 0x5F3759DF, jnp.int32)
            - lax.shift_right_logical(
                lax.bitcast_convert_type(sv, jnp.int32), 1),
            jnp.float32)
        y = seed
        for _ in range(4):
            y = y * (1.5 - 0.5 * sv * y * y)
        uni = jnp.full((16,), np.sqrt(K / N), jnp.float32)
        pos = sv > 0

        def scale_m(m, _):
            o = out_buf[pl.ds(m * 16, 16)]
            out_buf[pl.ds(m * 16, 16)] = jnp.where(pos, o * y, uni)
            return 0
        lax.fori_loop(0, N // 16, scale_m, 0)
        pltpu.sync_copy(out_buf, out_hbm)


def _to_blocked(x):
    return x.T.reshape(K, NCH, CH).transpose(1, 0, 2).reshape(NCH, CW)


def kernel(emb, qv, W1, b1, W2, b2):
    en = emb / (jnp.linalg.norm(emb, axis=1, keepdims=True) + 1e-12)
    en_pad = jnp.pad(en, ((0, NP - N), (0, 0)))
    emb_pad = jnp.pad(emb, ((0, NP - N), (0, 0)))
    b1e = (b1 + qv @ W1[D:]).reshape(1, D)
    W1a = W1[:D]
    W2p = jnp.pad(W2, ((0, 0), (0, D - K)))
    b2p = jnp.pad(b2, (0, D - K)).reshape(1, D)
    nbrs_p, g_p = _knn_coin(en_pad, emb_pad, W1a, b1e, W2p, b2p)
    nbrs = nbrs_p[:N]
    g = g_p[:N]

    kcol = jnp.arange(K, dtype=jnp.int32)[None, :]
    dest = (nbrs // CH) * CW + kcol * CH + nbrs % CH
    g_b = _to_blocked(g)
    idx_b = _to_blocked(dest)
    return _sc_walk(g_b, idx_b)
